# baseline (speedup 1.0000x reference)
import jax
import jax.numpy as jnp
from jax import lax
from jax.experimental import pallas as pl
from jax.experimental.pallas import tpu as pltpu
from jax.experimental.pallas import tpu_sc as plsc

TOPK = 3
N = 1000
B = 16384
RB = 1024

NC, NS, L = 2, 16, 16
NW = NC * NS
IDX_TOTAL = TOPK * B
IDX_PER_W = IDX_TOTAL // NW
VECS_PER_W = IDX_PER_W // L


def _topk_body(x_ref, ov_ref, oi_ref):
    xb = x_ref[...]
    R, Ncols = xb.shape
    fiota = lax.broadcasted_iota(jnp.int32, (R, Ncols), 1).astype(jnp.float32)
    neg = jnp.float32(-jnp.inf)
    big = jnp.float32(2048.0)
    vals = []
    idxs = []
    cur = xb
    for k in range(TOPK):
        v = jnp.max(cur, axis=1)
        i = jnp.min(jnp.where(cur == v[:, None], fiota, big), axis=1)
        vals.append(v)
        idxs.append(i)
        if k < TOPK - 1:
            cur = jnp.where(fiota == i[:, None], neg, cur)
    ov_ref[...] = jnp.stack(vals, axis=0)
    oi_ref[...] = jnp.stack(idxs, axis=0).astype(jnp.int32)


def _gather_body(idx_hbm, lbl_hbm, out_hbm, idxv, lblv, outv, sem):
    wid = lax.axis_index("s") * NC + lax.axis_index("c")
    pltpu.sync_copy(lbl_hbm, lblv)
    base = pl.multiple_of(wid * IDX_PER_W, 8)
    pltpu.sync_copy(idx_hbm.at[pl.ds(base, IDX_PER_W)], idxv)

    def vec_body(t, _):
        ivec = idxv[pl.ds(t * L, L)]
        outv[pl.ds(t * L, L)] = plsc.load_gather(lblv, [ivec])
        return 0

    lax.fori_loop(0, VECS_PER_W, vec_body, 0, unroll=8)
    pltpu.sync_copy(outv, out_hbm.at[pl.ds(base, IDX_PER_W)])


@jax.jit
def kernel(x, label_ids):
    vals_t, idx_t = pl.pallas_call(
        _topk_body,
        grid=(B // RB,),
        in_specs=[pl.BlockSpec((RB, N), lambda i: (i, 0))],
        out_specs=[
            pl.BlockSpec((TOPK, RB), lambda i: (0, i)),
            pl.BlockSpec((TOPK, RB), lambda i: (0, i)),
        ],
        out_shape=[
            jax.ShapeDtypeStruct((TOPK, B), jnp.float32),
            jax.ShapeDtypeStruct((TOPK, B), jnp.int32),
        ],
    )(x)

    mesh = plsc.VectorSubcoreMesh(
        core_axis_name="c", subcore_axis_name="s", num_cores=NC, num_subcores=NS
    )
    gather = pl.kernel(
        _gather_body,
        out_type=[jax.ShapeDtypeStruct((IDX_TOTAL,), jnp.int32)],
        mesh=mesh,
        compiler_params=pltpu.CompilerParams(needs_layout_passes=False),
        scratch_types=[
            pltpu.VMEM((IDX_PER_W,), jnp.int32),
            pltpu.VMEM((N,), jnp.int32),
            pltpu.VMEM((IDX_PER_W,), jnp.int32),
            pltpu.SemaphoreType.DMA,
        ],
    )
    (labels_flat,) = gather(idx_t.reshape(-1), label_ids)
    return vals_t.T, labels_flat.reshape(TOPK, B).T
